# baseline (device time: 128163 ns/iter reference)
import functools
import os

import jax
import jax.numpy as jnp
from jax import lax
from jax.experimental import pallas as pl
from jax.experimental.pallas import tpu as pltpu

MESH = pl.DeviceIdType.MESH
_COMPUTE_ONLY = bool(int(os.environ.get("KERNEL_COMPUTE_ONLY", "0")))


def kernel(x, dy):
    K, D = x.shape
    _, F = dy.shape
    G = F // 4
    H = D // 2
    T = 4
    TW = G // T
    HW = TW // 2
    U = 2 * T

    def body(x_hbm, dy_hbm, out_hbm, ld, xb, dyb, pk, zs, zr, sb, gx, gy,
             hx, hy, cb, ld_sem, zs_s, zr_s, axs, axr, ays, ayr,
             bxs, bxr, bys, byr, sts, stc):
        mx = lax.axis_index("x")
        my = lax.axis_index("y")
        mz = lax.axis_index("z")
        g = 2 * mx + my
        gp = 2 * (1 - mx) + my
        hh = 2 * mx + (1 - my)
        hp = 2 * (1 - mx) + (1 - my)
        xpeer = (1 - mx, my, mz)
        ypeer = (mx, 1 - my, mz)
        zpeer = (mx, my, 1 - mz)
        half0 = (1 - mz) * H
        half1 = mz * H

        load_plan = [("x", half0), ("x", half0 + TW), ("dy", 0), ("dy", 1),
                     ("x", half1), ("x", half1 + TW), ("dy", 2), ("dy", 3)]

        def start_load(c):
            kind, col = load_plan[c]
            src_col = col if kind == "x" else g * G + col * TW
            cp = pltpu.make_async_copy(
                (x_hbm if kind == "x" else dy_hbm).at[:, pl.ds(src_col, TW)],
                ld.at[c % 2], ld_sem.at[c % 2])
            cp.start()
            return cp

        pending = {0: start_load(0)}

        def finish_load(c):
            pending.pop(c).wait()
            if c + 1 < len(load_plan):
                pending[c + 1] = start_load(c + 1)
            kind, col = load_plan[c]
            val = ld[c % 2].astype(jnp.bfloat16)
            if kind == "x":
                xb[:, pl.ds(col, TW)] = val
            else:
                dyb[col % 2] = val

        if not _COMPUTE_ONLY:
            bar = pltpu.get_barrier_semaphore()
            for dev in (xpeer, ypeer, zpeer):
                pl.semaphore_signal(bar, inc=1, device_id=dev,
                                    device_id_type=MESH)
            pl.semaphore_wait(bar, 3)

        dn = (((0,), (0,)), ((), ()))

        def send_mm(t):
            p = lax.dot_general(
                xb[:, pl.ds(half0, H)], dyb[t % 2], dn,
                preferred_element_type=jnp.float32).astype(jnp.bfloat16)
            rs = []
            for u in (2 * t, 2 * t + 1):
                zs[u] = p[:, (u % 2) * HW:(u % 2) * HW + HW]
                if not _COMPUTE_ONLY:
                    r = pltpu.make_async_remote_copy(
                        zs.at[u], zr.at[u], zs_s.at[u], zr_s.at[u],
                        device_id=zpeer, device_id_type=MESH)
                    r.start()
                    rs.append(r)
            return rs

        def keep_mm(t):
            pk[t] = lax.dot_general(
                xb[:, pl.ds(half1, H)], dyb[t % 2], dn,
                preferred_element_type=jnp.float32)

        zrd = {}
        finish_load(0)
        finish_load(1)
        finish_load(2)
        zrd[0] = send_mm(0)
        finish_load(3)
        zrd[1] = send_mm(1)
        finish_load(4)
        finish_load(5)
        keep_mm(0)
        keep_mm(1)
        finish_load(6)
        zrd[2] = send_mm(2)
        keep_mm(2)
        finish_load(7)
        zrd[3] = send_mm(3)
        keep_mm(3)

        stcp = [None, None]
        cb_uses = [0]

        def store_via_cb(val_bf16, out_col):
            slot = cb_uses[0] % 2
            if stcp[slot] is not None:
                stcp[slot].wait()
            cb[slot] = val_bf16.astype(jnp.float32)
            cp = pltpu.make_async_copy(
                cb.at[slot], out_hbm.at[:, pl.ds(out_col, TW)], stc.at[slot])
            cp.start()
            stcp[slot] = cp
            cb_uses[0] += 1

        axd, ayd, std = {}, {}, []
        for u in range(U):
            t, i = u // 2, u % 2
            if not _COMPUTE_ONLY:
                zrd[t][i].wait()
            s = pk[t][:, i * HW:i * HW + HW] + (
                (zs if _COMPUTE_ONLY else zr)[u].astype(jnp.float32))
            pk[t, :, i * HW:i * HW + HW] = s
            sb[u] = s.astype(jnp.bfloat16)
            if not _COMPUTE_ONLY:
                ax = pltpu.make_async_remote_copy(
                    sb.at[u], gx.at[u], axs.at[u], axr.at[u],
                    device_id=xpeer, device_id_type=MESH)
                ax.start()
                axd[u] = ax
                ay = pltpu.make_async_remote_copy(
                    sb.at[u], gy.at[u], ays.at[u], ayr.at[u],
                    device_id=ypeer, device_id_type=MESH)
                ay.start()
                ayd[u] = ay
            if i == 1:
                st = pltpu.make_async_copy(
                    pk.at[t], out_hbm.at[:, pl.ds(g * G + t * TW, TW)],
                    sts.at[t])
                st.start()
                std.append(st)

        bxd, byd = [], []
        gxr = sb if _COMPUTE_ONLY else gx
        gyr = sb if _COMPUTE_ONLY else gy
        for t in range(T):
            if not _COMPUTE_ONLY:
                axd[2 * t + 1].wait()
                by = pltpu.make_async_remote_copy(
                    gx.at[2 * t + 1], hy.at[t], bys.at[t], byr.at[t],
                    device_id=ypeer, device_id_type=MESH)
                by.start()
                byd.append(by)
                ayd[2 * t].wait()
                bx = pltpu.make_async_remote_copy(
                    gy.at[2 * t], hx.at[t], bxs.at[t], bxr.at[t],
                    device_id=xpeer, device_id_type=MESH)
                bx.start()
                bxd.append(bx)
                axd[2 * t].wait()
                ayd[2 * t + 1].wait()
            store_via_cb(
                jnp.concatenate([gxr[2 * t], gxr[2 * t + 1]], axis=1),
                gp * G + t * TW)
            store_via_cb(
                jnp.concatenate([gyr[2 * t], gyr[2 * t + 1]], axis=1),
                hh * G + t * TW)

        for t in range(T):
            if not _COMPUTE_ONLY:
                bxd[t].wait()
                byd[t].wait()
                hx_t, hy_t = hx[t], hy[t]
            else:
                hx_t, hy_t = sb[2 * t], sb[2 * t + 1]
            store_via_cb(
                jnp.concatenate([hx_t, hy_t], axis=1), hp * G + t * TW)

        for st in std:
            st.wait()
        for cp in stcp:
            if cp is not None:
                cp.wait()

        if not _COMPUTE_ONLY:
            @functools.partial(pl.run_scoped,
                               sem2=pltpu.SemaphoreType.REGULAR)
            def _(sem2):
                for dev in (xpeer, ypeer, zpeer):
                    pl.semaphore_signal(sem2, inc=1, device_id=dev,
                                        device_id_type=MESH)
                pl.semaphore_wait(sem2, 3)

    return pl.pallas_call(
        body,
        out_shape=jax.ShapeDtypeStruct((H, F), jnp.float32),
        in_specs=[
            pl.BlockSpec(memory_space=pl.ANY),
            pl.BlockSpec(memory_space=pl.ANY),
        ],
        out_specs=pl.BlockSpec(memory_space=pl.ANY),
        scratch_shapes=[
            pltpu.VMEM((2, K, TW), jnp.float32),
            pltpu.VMEM((K, D), jnp.bfloat16),
            pltpu.VMEM((2, K, TW), jnp.bfloat16),
            pltpu.VMEM((T, H, TW), jnp.float32),
            pltpu.VMEM((U, H, HW), jnp.bfloat16),
            pltpu.VMEM((U, H, HW), jnp.bfloat16),
            pltpu.VMEM((U, H, HW), jnp.bfloat16),
            pltpu.VMEM((U, H, HW), jnp.bfloat16),
            pltpu.VMEM((U, H, HW), jnp.bfloat16),
            pltpu.VMEM((T, H, HW), jnp.bfloat16),
            pltpu.VMEM((T, H, HW), jnp.bfloat16),
            pltpu.VMEM((2, H, TW), jnp.float32),
            pltpu.SemaphoreType.DMA((2,)),
            pltpu.SemaphoreType.DMA((U,)),
            pltpu.SemaphoreType.DMA((U,)),
            pltpu.SemaphoreType.DMA((U,)),
            pltpu.SemaphoreType.DMA((U,)),
            pltpu.SemaphoreType.DMA((U,)),
            pltpu.SemaphoreType.DMA((U,)),
            pltpu.SemaphoreType.DMA((T,)),
            pltpu.SemaphoreType.DMA((T,)),
            pltpu.SemaphoreType.DMA((T,)),
            pltpu.SemaphoreType.DMA((T,)),
            pltpu.SemaphoreType.DMA((T,)),
            pltpu.SemaphoreType.DMA((2,)),
        ],
        compiler_params=pltpu.CompilerParams(
            collective_id=None if _COMPUTE_ONLY else 0,
            vmem_limit_bytes=100 * 1024 * 1024,
        ),
    )(x, dy)


# device time: 59302 ns/iter; 2.1612x vs baseline; 2.1612x over previous
import functools
import os

import jax
import jax.numpy as jnp
from jax import lax
from jax.experimental import pallas as pl
from jax.experimental.pallas import tpu as pltpu

MESH = pl.DeviceIdType.MESH
_COMPUTE_ONLY = bool(int(os.environ.get("KERNEL_COMPUTE_ONLY", "0")))
_NO_CID = bool(int(os.environ.get("KERNEL_NO_CID", "0")))
_FUSE_TLHS = bool(int(os.environ.get("KERNEL_FUSE_TLHS", "0")))


def kernel(x, dy):
    K, D = x.shape
    _, F = dy.shape
    G = F // 4
    H = D // 2
    T = 4
    TW = G // T
    HW = TW // 2
    U = 2 * T

    def body(x_hbm, dy_hbm, out_hbm, ld, xb, dyb, pk, zs, zr, sb, gx, gy,
             hx, hy, cb, ld_sem, zs_s, zr_s, axs, axr, ays, ayr,
             bxs, bxr, bys, byr, sts, stc):
        mx = lax.axis_index("x")
        my = lax.axis_index("y")
        mz = lax.axis_index("z")
        g = 2 * mx + my
        gp = 2 * (1 - mx) + my
        hh = 2 * mx + (1 - my)
        hp = 2 * (1 - mx) + (1 - my)
        xpeer = (1 - mx, my, mz)
        ypeer = (mx, 1 - my, mz)
        zpeer = (mx, my, 1 - mz)
        half0 = (1 - mz) * H
        half1 = mz * H

        load_plan = [("x", half0), ("x", half0 + TW), ("dy", 0), ("dy", 1),
                     ("x", half1), ("x", half1 + TW), ("dy", 2), ("dy", 3)]

        def start_load(c):
            kind, col = load_plan[c]
            src_col = col if kind == "x" else g * G + col * TW
            cp = pltpu.make_async_copy(
                (x_hbm if kind == "x" else dy_hbm).at[:, pl.ds(src_col, TW)],
                ld.at[c % 2], ld_sem.at[c % 2])
            cp.start()
            return cp

        pending = {0: start_load(0)}

        def finish_load(c):
            pending.pop(c).wait()
            if c + 1 < len(load_plan):
                pending[c + 1] = start_load(c + 1)
            kind, col = load_plan[c]
            val = ld[c % 2].astype(jnp.bfloat16)
            if kind == "x":
                xb[:, pl.ds(col, TW)] = val
            else:
                dyb[col % 2] = val

        if not _COMPUTE_ONLY and not _NO_CID:
            bar = pltpu.get_barrier_semaphore()
            for dev in (xpeer, ypeer, zpeer):
                pl.semaphore_signal(bar, inc=1, device_id=dev,
                                    device_id_type=MESH)
            pl.semaphore_wait(bar, 3)

        dn = (((0,), (0,)), ((), ()))

        def send_mm(t):
            p = lax.dot_general(
                xb[:, pl.ds(half0, H)], dyb[t % 2], dn,
                preferred_element_type=jnp.float32).astype(jnp.bfloat16)
            rs = []
            for u in (2 * t, 2 * t + 1):
                zs[u] = p[:, (u % 2) * HW:(u % 2) * HW + HW]
                if not _COMPUTE_ONLY:
                    r = pltpu.make_async_remote_copy(
                        zs.at[u], zr.at[u], zs_s.at[u], zr_s.at[u],
                        device_id=zpeer, device_id_type=MESH)
                    r.start()
                    rs.append(r)
            return rs

        def keep_mm(t):
            pk[t] = lax.dot_general(
                xb[:, pl.ds(half1, H)], dyb[t % 2], dn,
                preferred_element_type=jnp.float32)

        zrd = {}
        finish_load(0)
        finish_load(1)
        finish_load(2)
        zrd[0] = send_mm(0)
        finish_load(3)
        zrd[1] = send_mm(1)
        finish_load(4)
        finish_load(5)
        keep_mm(0)
        keep_mm(1)
        finish_load(6)
        zrd[2] = send_mm(2)
        keep_mm(2)
        finish_load(7)
        zrd[3] = send_mm(3)
        keep_mm(3)

        stcp = [None, None]
        cb_uses = [0]

        def store_via_cb(val_bf16, out_col):
            slot = cb_uses[0] % 2
            if stcp[slot] is not None:
                stcp[slot].wait()
            cb[slot] = val_bf16.astype(jnp.float32)
            cp = pltpu.make_async_copy(
                cb.at[slot], out_hbm.at[:, pl.ds(out_col, TW)], stc.at[slot])
            cp.start()
            stcp[slot] = cp
            cb_uses[0] += 1

        axd, ayd, std = {}, {}, []
        for u in range(U):
            t, i = u // 2, u % 2
            if not _COMPUTE_ONLY:
                zrd[t][i].wait()
            s = pk[t][:, i * HW:i * HW + HW] + (
                (zs if _COMPUTE_ONLY else zr)[u].astype(jnp.float32))
            pk[t, :, i * HW:i * HW + HW] = s
            sb[u] = s.astype(jnp.bfloat16)
            if not _COMPUTE_ONLY:
                ax = pltpu.make_async_remote_copy(
                    sb.at[u], gx.at[u], axs.at[u], axr.at[u],
                    device_id=xpeer, device_id_type=MESH)
                ax.start()
                axd[u] = ax
                ay = pltpu.make_async_remote_copy(
                    sb.at[u], gy.at[u], ays.at[u], ayr.at[u],
                    device_id=ypeer, device_id_type=MESH)
                ay.start()
                ayd[u] = ay
            if i == 1:
                st = pltpu.make_async_copy(
                    pk.at[t], out_hbm.at[:, pl.ds(g * G + t * TW, TW)],
                    sts.at[t])
                st.start()
                std.append(st)

        bxd, byd = [], []
        gxr = sb if _COMPUTE_ONLY else gx
        gyr = sb if _COMPUTE_ONLY else gy
        for t in range(T):
            if not _COMPUTE_ONLY:
                axd[2 * t + 1].wait()
                by = pltpu.make_async_remote_copy(
                    gx.at[2 * t + 1], hy.at[t], bys.at[t], byr.at[t],
                    device_id=ypeer, device_id_type=MESH)
                by.start()
                byd.append(by)
                ayd[2 * t].wait()
                bx = pltpu.make_async_remote_copy(
                    gy.at[2 * t], hx.at[t], bxs.at[t], bxr.at[t],
                    device_id=xpeer, device_id_type=MESH)
                bx.start()
                bxd.append(bx)
                axd[2 * t].wait()
                ayd[2 * t + 1].wait()
            store_via_cb(
                jnp.concatenate([gxr[2 * t], gxr[2 * t + 1]], axis=1),
                gp * G + t * TW)
            store_via_cb(
                jnp.concatenate([gyr[2 * t], gyr[2 * t + 1]], axis=1),
                hh * G + t * TW)

        for t in range(T):
            if not _COMPUTE_ONLY:
                bxd[t].wait()
                byd[t].wait()
                hx_t, hy_t = hx[t], hy[t]
            else:
                hx_t, hy_t = sb[2 * t], sb[2 * t + 1]
            store_via_cb(
                jnp.concatenate([hx_t, hy_t], axis=1), hp * G + t * TW)

        for st in std:
            st.wait()
        for cp in stcp:
            if cp is not None:
                cp.wait()

        if not _COMPUTE_ONLY:
            @functools.partial(pl.run_scoped,
                               sem2=pltpu.SemaphoreType.REGULAR)
            def _(sem2):
                for dev in (xpeer, ypeer, zpeer):
                    pl.semaphore_signal(sem2, inc=1, device_id=dev,
                                        device_id_type=MESH)
                pl.semaphore_wait(sem2, 3)

    return pl.pallas_call(
        body,
        out_shape=jax.ShapeDtypeStruct((H, F), jnp.float32),
        in_specs=[
            pl.BlockSpec(memory_space=pl.ANY),
            pl.BlockSpec(memory_space=pl.ANY),
        ],
        out_specs=pl.BlockSpec(memory_space=pl.ANY),
        scratch_shapes=[
            pltpu.VMEM((2, K, TW), jnp.float32),
            pltpu.VMEM((K, D), jnp.bfloat16),
            pltpu.VMEM((2, K, TW), jnp.bfloat16),
            pltpu.VMEM((T, H, TW), jnp.float32),
            pltpu.VMEM((U, H, HW), jnp.bfloat16),
            pltpu.VMEM((U, H, HW), jnp.bfloat16),
            pltpu.VMEM((U, H, HW), jnp.bfloat16),
            pltpu.VMEM((U, H, HW), jnp.bfloat16),
            pltpu.VMEM((U, H, HW), jnp.bfloat16),
            pltpu.VMEM((T, H, HW), jnp.bfloat16),
            pltpu.VMEM((T, H, HW), jnp.bfloat16),
            pltpu.VMEM((2, H, TW), jnp.float32),
            pltpu.SemaphoreType.DMA((2,)),
            pltpu.SemaphoreType.DMA((U,)),
            pltpu.SemaphoreType.DMA((U,)),
            pltpu.SemaphoreType.DMA((U,)),
            pltpu.SemaphoreType.DMA((U,)),
            pltpu.SemaphoreType.DMA((U,)),
            pltpu.SemaphoreType.DMA((U,)),
            pltpu.SemaphoreType.DMA((T,)),
            pltpu.SemaphoreType.DMA((T,)),
            pltpu.SemaphoreType.DMA((T,)),
            pltpu.SemaphoreType.DMA((T,)),
            pltpu.SemaphoreType.DMA((T,)),
            pltpu.SemaphoreType.DMA((2,)),
        ],
        compiler_params=pltpu.CompilerParams(
            collective_id=None if (_COMPUTE_ONLY or _NO_CID) else 0,
            vmem_limit_bytes=100 * 1024 * 1024,
            fuse_transposed_lhs_in_matmul=_FUSE_TLHS,
        ),
    )(x, dy)
